# Initial kernel scaffold; baseline (speedup 1.0000x reference)
import jax
import jax.numpy as jnp
from jax.experimental import pallas as pl


def kernel(positions, cell, neighbors, mask, offsets, atomic_numbers):
    raise NotImplementedError("write your pallas kernel here")


# speedup vs baseline: 19.5451x; 19.5451x over previous
import functools

import numpy as np
import jax
import jax.numpy as jnp
from jax import lax
from jax.experimental import pallas as pl
from jax.experimental.pallas import tpu as pltpu
from jax.experimental.pallas import tpu_sc as plsc

_RC = 5.0


def _num_workers():
    try:
        info = plsc.get_sparse_core_info()
        return info.num_cores * info.num_subcores
    except Exception:
        return 32


def _build_gather(B, A, N, NW, G):
    W = (B * A) // NW
    CH = (G * N) // 16
    mesh = plsc.VectorSubcoreMesh(core_axis_name="c", subcore_axis_name="s")

    @functools.partial(
        pl.kernel,
        out_type=[jax.ShapeDtypeStruct((B * A * N,), jnp.float32)] * 3,
        mesh=mesh,
        compiler_params=pltpu.CompilerParams(needs_layout_passes=False),
        scratch_types=[
            pltpu.VMEM((A * 3,), jnp.float32),
            pltpu.VMEM((G * N,), jnp.int32),
            pltpu.VMEM((G * N,), jnp.float32),
            pltpu.VMEM((G * N,), jnp.float32),
            pltpu.VMEM((G * N,), jnp.float32),
        ],
    )
    def k(pos_hbm, nb_hbm, gx_hbm, gy_hbm, gz_hbm, tab, nbv, gxs, gys, gzs):
        wid = lax.axis_index("s") * 2 + lax.axis_index("c")
        a0 = wid * W
        b = a0 // A
        pltpu.sync_copy(pos_hbm.at[b], tab)

        def round_body(r, carry):
            base = (a0 + r * G) * N
            pltpu.sync_copy(nb_hbm.at[pl.ds(base, G * N)], nbv)

            three = jnp.full((16,), 3, jnp.int32)
            one = jnp.full((16,), 1, jnp.int32)

            def chunk(c, carry2):
                p0 = c * 16
                i3 = nbv[pl.ds(p0, 16)] * three
                i3p1 = i3 + one
                i3p2 = i3p1 + one
                gxs[pl.ds(p0, 16)] = plsc.load_gather(tab, [i3])
                gys[pl.ds(p0, 16)] = plsc.load_gather(tab, [i3p1])
                gzs[pl.ds(p0, 16)] = plsc.load_gather(tab, [i3p2])
                return carry2

            lax.fori_loop(0, CH, chunk, 0)
            pltpu.sync_copy(gxs, gx_hbm.at[pl.ds(base, G * N)])
            pltpu.sync_copy(gys, gy_hbm.at[pl.ds(base, G * N)])
            pltpu.sync_copy(gzs, gz_hbm.at[pl.ds(base, G * N)])
            return carry

        lax.fori_loop(0, W // G, round_body, 0)

    return k


def _build_compute(BA, A, N, T):
    def body(gx_ref, gy_ref, gz_ref, ox_ref, oy_ref, oz_ref, ps_ref, cl_ref,
             mk_ref, rk_ref, cx_ref, cy_ref, cz_ref):
        gx, gy, gz = gx_ref[...], gy_ref[...], gz_ref[...]
        ox, oy, oz = ox_ref[...], oy_ref[...], oz_ref[...]
        px = ps_ref[:, 0:1]
        py = ps_ref[:, 1:2]
        pz = ps_ref[:, 2:3]
        cl = cl_ref[0]

        f32, bf16 = jnp.float32, jnp.bfloat16
        obx = ox.astype(bf16).astype(f32)
        oby = oy.astype(bf16).astype(f32)
        obz = oz.astype(bf16).astype(f32)
        clb = [cl[0, k].astype(bf16).astype(f32) for k in range(9)]
        ocx = obx * clb[0] + oby * clb[3] + obz * clb[6]
        ocy = obx * clb[1] + oby * clb[4] + obz * clb[7]
        ocz = obx * clb[2] + oby * clb[5] + obz * clb[8]
        dvx = (gx - px) + ocx
        dvy = (gy - py) + ocy
        dvz = (gz - pz) + ocz

        d = jnp.sqrt((dvx * dvx + dvy * dvy) + dvz * dvz)
        valid = mk_ref[...] != 0.0
        dsafe = jnp.where(valid, d, jnp.float32(1.0))
        cutf = (0.5 * (jnp.cos(dsafe * np.pi / _RC) + 1.0)
                * (dsafe < _RC).astype(jnp.float32))
        cut = jnp.where(valid, cutf / dsafe, jnp.float32(0.0))

        ii = lax.broadcasted_iota(jnp.int32, (N, N), 0)
        jj = lax.broadcasted_iota(jnp.int32, (N, N), 1)
        tri = ii < jj
        ident = (ii == jj).astype(jnp.float32)
        cutT = lax.dot_general(ident, cut, (((1,), (1,)), ((), ())),
                               preferred_element_type=jnp.float32,
                               precision=lax.Precision.HIGHEST)

        rows = []
        for t in range(T):
            col = cutT[:, t:t + 1]
            row = cut[t:t + 1, :]
            gtm = col > row
            tiem = (col == row) & tri
            cnt = jnp.logical_or(gtm, tiem).astype(jnp.float32)
            rows.append(jnp.sum(cnt, axis=0, keepdims=True))
        rank = jnp.concatenate(rows, axis=0)

        rk_ref[...] = rank.astype(jnp.int32)
        cx_ref[...] = cut * dvx
        cy_ref[...] = cut * dvy
        cz_ref[...] = cut * dvz

    tile = pl.BlockSpec((T, N), lambda i: (i, 0))
    return pl.pallas_call(
        body,
        grid=(BA // T,),
        in_specs=[tile, tile, tile, tile, tile, tile,
                  pl.BlockSpec((T, 3), lambda i: (i, 0)),
                  pl.BlockSpec((1, 1, 9), lambda i: ((i * T) // A, 0, 0)),
                  tile],
        out_specs=[tile, tile, tile, tile],
        out_shape=[jax.ShapeDtypeStruct((BA, N), jnp.int32)] +
                  [jax.ShapeDtypeStruct((BA, N), jnp.float32)] * 3,
    )


def _build_scatter(B, A, N, NW, G):
    W = (B * A) // NW
    CH = (G * N) // 16
    mesh = plsc.VectorSubcoreMesh(core_axis_name="c", subcore_axis_name="s")

    @functools.partial(
        pl.kernel,
        out_type=jax.ShapeDtypeStruct((B * A * N * 3,), jnp.float32),
        mesh=mesh,
        compiler_params=pltpu.CompilerParams(needs_layout_passes=False),
        scratch_types=[
            pltpu.VMEM((G * N,), jnp.int32),
            pltpu.VMEM((G * N,), jnp.float32),
            pltpu.VMEM((G * N,), jnp.float32),
            pltpu.VMEM((G * N,), jnp.float32),
            pltpu.VMEM((G * N * 3,), jnp.float32),
        ],
    )
    def k(rk_hbm, cx_hbm, cy_hbm, cz_hbm, out_hbm, rkv, cxv, cyv, czv, outv):
        wid = lax.axis_index("s") * 2 + lax.axis_index("c")
        a0 = wid * W
        lane = lax.broadcasted_iota(jnp.int32, (16,), 0)

        def round_body(r, carry):
            base = (a0 + r * G) * N
            pltpu.sync_copy(rk_hbm.at[pl.ds(base, G * N)], rkv)
            pltpu.sync_copy(cx_hbm.at[pl.ds(base, G * N)], cxv)
            pltpu.sync_copy(cy_hbm.at[pl.ds(base, G * N)], cyv)
            pltpu.sync_copy(cz_hbm.at[pl.ds(base, G * N)], czv)

            three = jnp.full((16,), 3, jnp.int32)
            one = jnp.full((16,), 1, jnp.int32)
            nvec = jnp.full((16,), N, jnp.int32)
            n3vec = jnp.full((16,), 3 * N, jnp.int32)

            def chunk(c, carry2):
                p0 = c * 16
                pos = lane + jnp.full((16,), p0, jnp.int32)
                aoff = lax.div(pos, nvec) * n3vec
                i3 = rkv[pl.ds(p0, 16)] * three + aoff
                i3p1 = i3 + one
                i3p2 = i3p1 + one
                plsc.store_scatter(outv, [i3], cxv[pl.ds(p0, 16)])
                plsc.store_scatter(outv, [i3p1], cyv[pl.ds(p0, 16)])
                plsc.store_scatter(outv, [i3p2], czv[pl.ds(p0, 16)])
                return carry2

            lax.fori_loop(0, CH, chunk, 0)
            pltpu.sync_copy(outv, out_hbm.at[pl.ds(base * 3, G * N * 3)])
            return carry

        lax.fori_loop(0, W // G, round_body, 0)

    return k


def kernel(positions, cell, neighbors, mask, offsets, atomic_numbers):
    B, A, N = neighbors.shape
    BA = B * A
    NW = _num_workers()
    G = 16
    T = 8
    assert (BA % NW == 0) and (A % ((BA) // NW) == 0 or ((BA // NW) % A == 0))
    assert ((BA // NW) % G == 0) and (G * N) % 16 == 0 and BA % T == 0

    pos_flat = positions.reshape(B, A * 3)
    nb_flat = neighbors.reshape(B * A * N)
    gx, gy, gz = _build_gather(B, A, N, NW, G)(pos_flat, nb_flat)

    off_p = jnp.moveaxis(offsets, 3, 0)
    rank, cx, cy, cz = _build_compute(BA, A, N, T)(
        gx.reshape(BA, N), gy.reshape(BA, N), gz.reshape(BA, N),
        off_p[0].reshape(BA, N), off_p[1].reshape(BA, N),
        off_p[2].reshape(BA, N),
        positions.reshape(BA, 3), cell.reshape(B, 1, 9), mask.reshape(BA, N))

    out = _build_scatter(B, A, N, NW, G)(
        rank.reshape(-1), cx.reshape(-1), cy.reshape(-1), cz.reshape(-1))
    return out.reshape(B, A, 3 * N)


# speedup vs baseline: 22.1764x; 1.1346x over previous
import functools

import numpy as np
import jax
import jax.numpy as jnp
from jax import lax
from jax.experimental import pallas as pl
from jax.experimental.pallas import tpu as pltpu
from jax.experimental.pallas import tpu_sc as plsc

_RC = 5.0


def _num_workers():
    try:
        info = plsc.get_sparse_core_info()
        return info.num_cores * info.num_subcores
    except Exception:
        return 32


def _build_gather(B, A, N, NW, G):
    W = (B * A) // NW
    CH = (G * N) // 16
    mesh = plsc.VectorSubcoreMesh(core_axis_name="c", subcore_axis_name="s")

    @functools.partial(
        pl.kernel,
        out_type=[jax.ShapeDtypeStruct((B * A * N,), jnp.float32)] * 3,
        mesh=mesh,
        compiler_params=pltpu.CompilerParams(needs_layout_passes=False),
        scratch_types=[
            pltpu.VMEM((A * 3,), jnp.float32),
            pltpu.VMEM((G * N,), jnp.int32),
            pltpu.VMEM((G * N,), jnp.float32),
            pltpu.VMEM((G * N,), jnp.float32),
            pltpu.VMEM((G * N,), jnp.float32),
        ],
    )
    def k(pos_hbm, nb_hbm, gx_hbm, gy_hbm, gz_hbm, tab, nbv, gxs, gys, gzs):
        wid = lax.axis_index("s") * 2 + lax.axis_index("c")
        a0 = wid * W
        b = a0 // A
        pltpu.sync_copy(pos_hbm.at[b], tab)

        def round_body(r, carry):
            base = (a0 + r * G) * N
            pltpu.sync_copy(nb_hbm.at[pl.ds(base, G * N)], nbv)

            three = jnp.full((16,), 3, jnp.int32)
            one = jnp.full((16,), 1, jnp.int32)

            def chunk(c, carry2):
                p0 = c * 16
                i3 = nbv[pl.ds(p0, 16)] * three
                i3p1 = i3 + one
                i3p2 = i3p1 + one
                gxs[pl.ds(p0, 16)] = plsc.load_gather(tab, [i3])
                gys[pl.ds(p0, 16)] = plsc.load_gather(tab, [i3p1])
                gzs[pl.ds(p0, 16)] = plsc.load_gather(tab, [i3p2])
                return carry2

            lax.fori_loop(0, CH, chunk, 0)
            pltpu.sync_copy(gxs, gx_hbm.at[pl.ds(base, G * N)])
            pltpu.sync_copy(gys, gy_hbm.at[pl.ds(base, G * N)])
            pltpu.sync_copy(gzs, gz_hbm.at[pl.ds(base, G * N)])
            return carry

        lax.fori_loop(0, W // G, round_body, 0)

    return k


def _build_compute(BA, A, N, T):
    def body(gx_ref, gy_ref, gz_ref, ox_ref, oy_ref, oz_ref, ps_ref, cl_ref,
             mk_ref, rk_ref, cx_ref, cy_ref, cz_ref):
        gx, gy, gz = gx_ref[...], gy_ref[...], gz_ref[...]
        ox, oy, oz = ox_ref[...], oy_ref[...], oz_ref[...]
        px = ps_ref[:, 0:1]
        py = ps_ref[:, 1:2]
        pz = ps_ref[:, 2:3]
        cl = cl_ref[0]

        f32, bf16 = jnp.float32, jnp.bfloat16
        obx = ox.astype(bf16).astype(f32)
        oby = oy.astype(bf16).astype(f32)
        obz = oz.astype(bf16).astype(f32)
        clb = [cl[0, k].astype(bf16).astype(f32) for k in range(9)]
        ocx = obx * clb[0] + oby * clb[3] + obz * clb[6]
        ocy = obx * clb[1] + oby * clb[4] + obz * clb[7]
        ocz = obx * clb[2] + oby * clb[5] + obz * clb[8]
        dvx = (gx - px) + ocx
        dvy = (gy - py) + ocy
        dvz = (gz - pz) + ocz

        d = jnp.sqrt((dvx * dvx + dvy * dvy) + dvz * dvz)
        valid = mk_ref[...] != 0.0
        dsafe = jnp.where(valid, d, jnp.float32(1.0))
        cutf = (0.5 * (jnp.cos(dsafe * np.pi / _RC) + 1.0)
                * (dsafe < _RC).astype(jnp.float32))
        cut = jnp.where(valid, cutf / dsafe, jnp.float32(0.0))

        ii = lax.broadcasted_iota(jnp.int32, (N, N), 0)
        jj = lax.broadcasted_iota(jnp.int32, (N, N), 1)
        tri = ii < jj
        ident = (ii == jj).astype(jnp.float32)
        cutT = lax.dot_general(ident, cut, (((1,), (1,)), ((), ())),
                               preferred_element_type=jnp.float32,
                               precision=lax.Precision.HIGHEST)

        ones_row = jnp.ones((1, N), jnp.float32)
        rows = []
        for t in range(T):
            col = cutT[:, t:t + 1]
            row = cut[t:t + 1, :]
            gtm = col > row
            tiem = (col == row) & tri
            cnt = jnp.logical_or(gtm, tiem).astype(jnp.float32)
            rows.append(lax.dot_general(ones_row, cnt, (((1,), (0,)), ((), ())),
                                        preferred_element_type=jnp.float32))
        rank = jnp.concatenate(rows, axis=0)

        rk_ref[...] = rank.astype(jnp.int32)
        cx_ref[...] = cut * dvx
        cy_ref[...] = cut * dvy
        cz_ref[...] = cut * dvz

    tile = pl.BlockSpec((T, N), lambda i: (i, 0))
    return pl.pallas_call(
        body,
        grid=(BA // T,),
        in_specs=[tile, tile, tile, tile, tile, tile,
                  pl.BlockSpec((T, 3), lambda i: (i, 0)),
                  pl.BlockSpec((1, 1, 9), lambda i: ((i * T) // A, 0, 0)),
                  tile],
        out_specs=[tile, tile, tile, tile],
        out_shape=[jax.ShapeDtypeStruct((BA, N), jnp.int32)] +
                  [jax.ShapeDtypeStruct((BA, N), jnp.float32)] * 3,
    )


def _build_scatter(B, A, N, NW, G):
    W = (B * A) // NW
    CH = (G * N) // 16
    mesh = plsc.VectorSubcoreMesh(core_axis_name="c", subcore_axis_name="s")

    @functools.partial(
        pl.kernel,
        out_type=jax.ShapeDtypeStruct((B * A * N * 3,), jnp.float32),
        mesh=mesh,
        compiler_params=pltpu.CompilerParams(needs_layout_passes=False),
        scratch_types=[
            pltpu.VMEM((G * N,), jnp.int32),
            pltpu.VMEM((G * N,), jnp.float32),
            pltpu.VMEM((G * N,), jnp.float32),
            pltpu.VMEM((G * N,), jnp.float32),
            pltpu.VMEM((G * N * 3,), jnp.float32),
        ],
    )
    def k(rk_hbm, cx_hbm, cy_hbm, cz_hbm, out_hbm, rkv, cxv, cyv, czv, outv):
        wid = lax.axis_index("s") * 2 + lax.axis_index("c")
        a0 = wid * W
        lane = lax.broadcasted_iota(jnp.int32, (16,), 0)

        def round_body(r, carry):
            base = (a0 + r * G) * N
            pltpu.sync_copy(rk_hbm.at[pl.ds(base, G * N)], rkv)
            pltpu.sync_copy(cx_hbm.at[pl.ds(base, G * N)], cxv)
            pltpu.sync_copy(cy_hbm.at[pl.ds(base, G * N)], cyv)
            pltpu.sync_copy(cz_hbm.at[pl.ds(base, G * N)], czv)

            three = jnp.full((16,), 3, jnp.int32)
            one = jnp.full((16,), 1, jnp.int32)
            nvec = jnp.full((16,), N, jnp.int32)
            n3vec = jnp.full((16,), 3 * N, jnp.int32)

            def chunk(c, carry2):
                p0 = c * 16
                pos = lane + jnp.full((16,), p0, jnp.int32)
                aoff = lax.div(pos, nvec) * n3vec
                i3 = rkv[pl.ds(p0, 16)] * three + aoff
                i3p1 = i3 + one
                i3p2 = i3p1 + one
                plsc.store_scatter(outv, [i3], cxv[pl.ds(p0, 16)])
                plsc.store_scatter(outv, [i3p1], cyv[pl.ds(p0, 16)])
                plsc.store_scatter(outv, [i3p2], czv[pl.ds(p0, 16)])
                return carry2

            lax.fori_loop(0, CH, chunk, 0)
            pltpu.sync_copy(outv, out_hbm.at[pl.ds(base * 3, G * N * 3)])
            return carry

        lax.fori_loop(0, W // G, round_body, 0)

    return k


def kernel(positions, cell, neighbors, mask, offsets, atomic_numbers):
    B, A, N = neighbors.shape
    BA = B * A
    NW = _num_workers()
    G = 64
    T = 16
    assert (BA % NW == 0) and (A % ((BA) // NW) == 0 or ((BA // NW) % A == 0))
    assert ((BA // NW) % G == 0) and (G * N) % 16 == 0 and BA % T == 0

    pos_flat = positions.reshape(B, A * 3)
    nb_flat = neighbors.reshape(B * A * N)
    gx, gy, gz = _build_gather(B, A, N, NW, G)(pos_flat, nb_flat)

    off_p = jnp.moveaxis(offsets, 3, 0)
    rank, cx, cy, cz = _build_compute(BA, A, N, T)(
        gx.reshape(BA, N), gy.reshape(BA, N), gz.reshape(BA, N),
        off_p[0].reshape(BA, N), off_p[1].reshape(BA, N),
        off_p[2].reshape(BA, N),
        positions.reshape(BA, 3), cell.reshape(B, 1, 9), mask.reshape(BA, N))

    out = _build_scatter(B, A, N, NW, G)(
        rank.reshape(-1), cx.reshape(-1), cy.reshape(-1), cz.reshape(-1))
    return out.reshape(B, A, 3 * N)


# speedup vs baseline: 22.6944x; 1.0234x over previous
import functools

import numpy as np
import jax
import jax.numpy as jnp
from jax import lax
from jax.experimental import pallas as pl
from jax.experimental.pallas import tpu as pltpu
from jax.experimental.pallas import tpu_sc as plsc

_RC = 5.0


def _num_workers():
    try:
        info = plsc.get_sparse_core_info()
        return info.num_cores * info.num_subcores
    except Exception:
        return 32


def _build_gather(B, A, N, NW, G):
    W = (B * A) // NW
    CH = (G * N) // 16
    mesh = plsc.VectorSubcoreMesh(core_axis_name="c", subcore_axis_name="s")

    @functools.partial(
        pl.kernel,
        out_type=[jax.ShapeDtypeStruct((B * A * N,), jnp.float32)] * 3,
        mesh=mesh,
        compiler_params=pltpu.CompilerParams(needs_layout_passes=False),
        scratch_types=[
            pltpu.VMEM((A * 3,), jnp.float32),
            pltpu.VMEM((G * N,), jnp.int32),
            pltpu.VMEM((G * N,), jnp.float32),
            pltpu.VMEM((G * N,), jnp.float32),
            pltpu.VMEM((G * N,), jnp.float32),
        ],
    )
    def k(pos_hbm, nb_hbm, gx_hbm, gy_hbm, gz_hbm, tab, nbv, gxs, gys, gzs):
        wid = lax.axis_index("s") * 2 + lax.axis_index("c")
        a0 = wid * W
        b = a0 // A
        pltpu.sync_copy(pos_hbm.at[b], tab)

        def round_body(r, carry):
            base = (a0 + r * G) * N
            pltpu.sync_copy(nb_hbm.at[pl.ds(base, G * N)], nbv)

            three = jnp.full((16,), 3, jnp.int32)
            one = jnp.full((16,), 1, jnp.int32)

            def chunk(c, carry2):
                p0 = c * 16
                i3 = nbv[pl.ds(p0, 16)] * three
                i3p1 = i3 + one
                i3p2 = i3p1 + one
                gxs[pl.ds(p0, 16)] = plsc.load_gather(tab, [i3])
                gys[pl.ds(p0, 16)] = plsc.load_gather(tab, [i3p1])
                gzs[pl.ds(p0, 16)] = plsc.load_gather(tab, [i3p2])
                return carry2

            lax.fori_loop(0, CH, chunk, 0)
            pltpu.sync_copy(gxs, gx_hbm.at[pl.ds(base, G * N)])
            pltpu.sync_copy(gys, gy_hbm.at[pl.ds(base, G * N)])
            pltpu.sync_copy(gzs, gz_hbm.at[pl.ds(base, G * N)])
            return carry

        lax.fori_loop(0, W // G, round_body, 0)

    return k


def _build_compute(BA, A, N, T):
    def body(gx_ref, gy_ref, gz_ref, ox_ref, oy_ref, oz_ref, ps_ref, cl_ref,
             mk_ref, rk_ref, cx_ref, cy_ref, cz_ref):
        gx, gy, gz = gx_ref[...], gy_ref[...], gz_ref[...]
        ox, oy, oz = ox_ref[...], oy_ref[...], oz_ref[...]
        px = ps_ref[:, 0:1]
        py = ps_ref[:, 1:2]
        pz = ps_ref[:, 2:3]
        cl = cl_ref[0]

        f32, bf16 = jnp.float32, jnp.bfloat16
        obx = ox.astype(bf16).astype(f32)
        oby = oy.astype(bf16).astype(f32)
        obz = oz.astype(bf16).astype(f32)
        clb = [cl[0, k].astype(bf16).astype(f32) for k in range(9)]
        ocx = obx * clb[0] + oby * clb[3] + obz * clb[6]
        ocy = obx * clb[1] + oby * clb[4] + obz * clb[7]
        ocz = obx * clb[2] + oby * clb[5] + obz * clb[8]
        dvx = (gx - px) + ocx
        dvy = (gy - py) + ocy
        dvz = (gz - pz) + ocz

        d = jnp.sqrt((dvx * dvx + dvy * dvy) + dvz * dvz)
        valid = mk_ref[...] != 0.0
        dsafe = jnp.where(valid, d, jnp.float32(1.0))
        cutf = (0.5 * (jnp.cos(dsafe * np.pi / _RC) + 1.0)
                * (dsafe < _RC).astype(jnp.float32))
        cut = jnp.where(valid, cutf / dsafe, jnp.float32(0.0))

        ii = lax.broadcasted_iota(jnp.int32, (N, N), 0)
        jj = lax.broadcasted_iota(jnp.int32, (N, N), 1)
        tri = ii < jj
        ident = (ii == jj).astype(jnp.float32)
        cutT = lax.dot_general(ident, cut, (((1,), (1,)), ((), ())),
                               preferred_element_type=jnp.float32,
                               precision=lax.Precision.HIGHEST)

        ones_row = jnp.ones((1, N), jnp.float32)
        rows = []
        for t in range(T):
            col = cutT[:, t:t + 1]
            row = cut[t:t + 1, :]
            gtm = col > row
            tiem = (col == row) & tri
            cnt = jnp.logical_or(gtm, tiem).astype(jnp.float32)
            rows.append(lax.dot_general(ones_row, cnt, (((1,), (0,)), ((), ())),
                                        preferred_element_type=jnp.float32))
        rank = jnp.concatenate(rows, axis=0)

        rk_ref[...] = rank.astype(jnp.int32)
        cx_ref[...] = cut * dvx
        cy_ref[...] = cut * dvy
        cz_ref[...] = cut * dvz

    tile = pl.BlockSpec((T, N), lambda i: (i, 0))
    return pl.pallas_call(
        body,
        grid=(BA // T,),
        in_specs=[tile, tile, tile, tile, tile, tile,
                  pl.BlockSpec((T, 3), lambda i: (i, 0)),
                  pl.BlockSpec((1, 1, 9), lambda i: ((i * T) // A, 0, 0)),
                  tile],
        out_specs=[tile, tile, tile, tile],
        out_shape=[jax.ShapeDtypeStruct((BA, N), jnp.int32)] +
                  [jax.ShapeDtypeStruct((BA, N), jnp.float32)] * 3,
    )


def _build_scatter(B, A, N, NW, G):
    W = (B * A) // NW
    CH = (G * N) // 16
    mesh = plsc.VectorSubcoreMesh(core_axis_name="c", subcore_axis_name="s")

    @functools.partial(
        pl.kernel,
        out_type=jax.ShapeDtypeStruct((B * A * N * 3,), jnp.float32),
        mesh=mesh,
        compiler_params=pltpu.CompilerParams(needs_layout_passes=False),
        scratch_types=[
            pltpu.VMEM((G * N,), jnp.int32),
            pltpu.VMEM((G * N,), jnp.float32),
            pltpu.VMEM((G * N,), jnp.float32),
            pltpu.VMEM((G * N,), jnp.float32),
            pltpu.VMEM((G * N * 3,), jnp.float32),
        ],
    )
    def k(rk_hbm, cx_hbm, cy_hbm, cz_hbm, out_hbm, rkv, cxv, cyv, czv, outv):
        wid = lax.axis_index("s") * 2 + lax.axis_index("c")
        a0 = wid * W
        lane = lax.broadcasted_iota(jnp.int32, (16,), 0)

        def round_body(r, carry):
            base = (a0 + r * G) * N
            pltpu.sync_copy(rk_hbm.at[pl.ds(base, G * N)], rkv)
            pltpu.sync_copy(cx_hbm.at[pl.ds(base, G * N)], cxv)
            pltpu.sync_copy(cy_hbm.at[pl.ds(base, G * N)], cyv)
            pltpu.sync_copy(cz_hbm.at[pl.ds(base, G * N)], czv)

            three = jnp.full((16,), 3, jnp.int32)
            one = jnp.full((16,), 1, jnp.int32)
            nvec = jnp.full((16,), N, jnp.int32)
            n3vec = jnp.full((16,), 3 * N, jnp.int32)

            def chunk(c, carry2):
                p0 = c * 16
                pos = lane + jnp.full((16,), p0, jnp.int32)
                aoff = lax.div(pos, nvec) * n3vec
                i3 = rkv[pl.ds(p0, 16)] * three + aoff
                i3p1 = i3 + one
                i3p2 = i3p1 + one
                plsc.store_scatter(outv, [i3], cxv[pl.ds(p0, 16)])
                plsc.store_scatter(outv, [i3p1], cyv[pl.ds(p0, 16)])
                plsc.store_scatter(outv, [i3p2], czv[pl.ds(p0, 16)])
                return carry2

            lax.fori_loop(0, CH, chunk, 0)
            pltpu.sync_copy(outv, out_hbm.at[pl.ds(base * 3, G * N * 3)])
            return carry

        lax.fori_loop(0, W // G, round_body, 0)

    return k


def kernel(positions, cell, neighbors, mask, offsets, atomic_numbers):
    B, A, N = neighbors.shape
    BA = B * A
    NW = _num_workers()
    G = 64
    T = 16
    W = A // NW
    assert A % NW == 0 and W % G == 0 and (G * N) % 16 == 0 and A % T == 0
    del BA
    gather_k = _build_gather(1, A, N, NW, G)
    compute_k = _build_compute(A, A, N, T)
    scatter_k = _build_scatter(1, A, N, NW, G)
    off_p = jnp.moveaxis(offsets, 3, 0)

    outs = []
    for b in range(B):
        gx, gy, gz = gather_k(positions[b].reshape(1, A * 3),
                              neighbors[b].reshape(A * N))
        rank, cx, cy, cz = compute_k(
            gx.reshape(A, N), gy.reshape(A, N), gz.reshape(A, N),
            off_p[0, b], off_p[1, b], off_p[2, b],
            positions[b], cell[b].reshape(1, 1, 9), mask[b])
        outs.append(scatter_k(rank.reshape(-1), cx.reshape(-1),
                              cy.reshape(-1), cz.reshape(-1)))
    return jnp.stack(outs).reshape(B, A, 3 * N)


# speedup vs baseline: 34.9480x; 1.5399x over previous
import functools

import numpy as np
import jax
import jax.numpy as jnp
from jax import lax
from jax.experimental import pallas as pl
from jax.experimental.pallas import tpu as pltpu
from jax.experimental.pallas import tpu_sc as plsc

_RC = 5.0


def _num_workers():
    try:
        info = plsc.get_sparse_core_info()
        return info.num_cores * info.num_subcores
    except Exception:
        return 32


def _build_gather(B, A, N, NW, G):
    W = (B * A) // NW
    CH = (G * N) // 16
    mesh = plsc.VectorSubcoreMesh(core_axis_name="c", subcore_axis_name="s")

    @functools.partial(
        pl.kernel,
        out_type=[jax.ShapeDtypeStruct((B * A * N,), jnp.float32)] * 3,
        mesh=mesh,
        compiler_params=pltpu.CompilerParams(needs_layout_passes=False),
        scratch_types=[
            pltpu.VMEM((A * 3,), jnp.float32),
            pltpu.VMEM((G * N,), jnp.int32),
            pltpu.VMEM((G * N,), jnp.float32),
            pltpu.VMEM((G * N,), jnp.float32),
            pltpu.VMEM((G * N,), jnp.float32),
        ],
    )
    def k(pos_hbm, nb_hbm, gx_hbm, gy_hbm, gz_hbm, tab, nbv, gxs, gys, gzs):
        wid = lax.axis_index("s") * 2 + lax.axis_index("c")
        a0 = wid * W
        b = a0 // A
        pltpu.sync_copy(pos_hbm.at[b], tab)

        def round_body(r, carry):
            base = (a0 + r * G) * N
            pltpu.sync_copy(nb_hbm.at[pl.ds(base, G * N)], nbv)

            three = jnp.full((16,), 3, jnp.int32)
            one = jnp.full((16,), 1, jnp.int32)

            def chunk(c, carry2):
                p0 = c * 16
                i3 = nbv[pl.ds(p0, 16)] * three
                i3p1 = i3 + one
                i3p2 = i3p1 + one
                gxs[pl.ds(p0, 16)] = plsc.load_gather(tab, [i3])
                gys[pl.ds(p0, 16)] = plsc.load_gather(tab, [i3p1])
                gzs[pl.ds(p0, 16)] = plsc.load_gather(tab, [i3p2])
                return carry2

            lax.fori_loop(0, CH, chunk, 0)
            pltpu.sync_copy(gxs, gx_hbm.at[pl.ds(base, G * N)])
            pltpu.sync_copy(gys, gy_hbm.at[pl.ds(base, G * N)])
            pltpu.sync_copy(gzs, gz_hbm.at[pl.ds(base, G * N)])
            return carry

        lax.fori_loop(0, W // G, round_body, 0)

    return k


def _build_compute(BA, A, N, T):
    def body(gx_ref, gy_ref, gz_ref, ox_ref, oy_ref, oz_ref, ps_ref, cl_ref,
             mk_ref, rk_ref, cx_ref, cy_ref, cz_ref):
        gx, gy, gz = gx_ref[...], gy_ref[...], gz_ref[...]
        ox, oy, oz = ox_ref[...], oy_ref[...], oz_ref[...]
        px = ps_ref[:, 0:1]
        py = ps_ref[:, 1:2]
        pz = ps_ref[:, 2:3]
        cl = cl_ref[0]

        f32, bf16 = jnp.float32, jnp.bfloat16
        obx = ox.astype(bf16).astype(f32)
        oby = oy.astype(bf16).astype(f32)
        obz = oz.astype(bf16).astype(f32)
        clb = [cl[0, k].astype(bf16).astype(f32) for k in range(9)]
        ocx = obx * clb[0] + oby * clb[3] + obz * clb[6]
        ocy = obx * clb[1] + oby * clb[4] + obz * clb[7]
        ocz = obx * clb[2] + oby * clb[5] + obz * clb[8]
        dvx = (gx - px) + ocx
        dvy = (gy - py) + ocy
        dvz = (gz - pz) + ocz

        d = jnp.sqrt((dvx * dvx + dvy * dvy) + dvz * dvz)
        valid = mk_ref[...] != 0.0
        dsafe = jnp.where(valid, d, jnp.float32(1.0))
        cutf = (0.5 * (jnp.cos(dsafe * np.pi / _RC) + 1.0)
                * (dsafe < _RC).astype(jnp.float32))
        cut = jnp.where(valid, cutf / dsafe, jnp.float32(0.0))

        ii = lax.broadcasted_iota(jnp.int32, (N, N), 0)
        jj = lax.broadcasted_iota(jnp.int32, (N, N), 1)
        tri = ii < jj
        ident = (ii == jj).astype(jnp.float32)
        cutT = lax.dot_general(ident, cut, (((1,), (1,)), ((), ())),
                               preferred_element_type=jnp.float32,
                               precision=lax.Precision.HIGHEST)

        ones_row = jnp.ones((1, N), jnp.float32)
        rows = []
        for t in range(T):
            col = cutT[:, t:t + 1]
            row = cut[t:t + 1, :]
            gtm = col > row
            tiem = (col == row) & tri
            cnt = jnp.logical_or(gtm, tiem).astype(jnp.float32)
            rows.append(lax.dot_general(ones_row, cnt, (((1,), (0,)), ((), ())),
                                        preferred_element_type=jnp.float32))
        rank = jnp.concatenate(rows, axis=0)

        rk_ref[...] = rank.astype(jnp.int32)
        cx_ref[...] = cut * dvx
        cy_ref[...] = cut * dvy
        cz_ref[...] = cut * dvz

    tile = pl.BlockSpec((T, N), lambda i: (i, 0))
    return pl.pallas_call(
        body,
        grid=(BA // T,),
        in_specs=[tile, tile, tile, tile, tile, tile,
                  pl.BlockSpec((T, 3), lambda i: (i, 0)),
                  pl.BlockSpec((1, 1, 9), lambda i: ((i * T) // A, 0, 0)),
                  tile],
        out_specs=[tile, tile, tile, tile],
        out_shape=[jax.ShapeDtypeStruct((BA, N), jnp.int32)] +
                  [jax.ShapeDtypeStruct((BA, N), jnp.float32)] * 3,
    )


def _build_scatter(B, A, N, NW, G):
    W = (B * A) // NW
    CH = (G * N) // 16
    mesh = plsc.VectorSubcoreMesh(core_axis_name="c", subcore_axis_name="s")

    @functools.partial(
        pl.kernel,
        out_type=jax.ShapeDtypeStruct((B * A * N * 3,), jnp.float32),
        mesh=mesh,
        compiler_params=pltpu.CompilerParams(needs_layout_passes=False),
        scratch_types=[
            pltpu.VMEM((G * N,), jnp.int32),
            pltpu.VMEM((G * N,), jnp.float32),
            pltpu.VMEM((G * N,), jnp.float32),
            pltpu.VMEM((G * N,), jnp.float32),
            pltpu.VMEM((G * N * 3,), jnp.float32),
        ],
    )
    def k(rk_hbm, cx_hbm, cy_hbm, cz_hbm, out_hbm, rkv, cxv, cyv, czv, outv):
        wid = lax.axis_index("s") * 2 + lax.axis_index("c")
        a0 = wid * W
        lane = lax.broadcasted_iota(jnp.int32, (16,), 0)

        def round_body(r, carry):
            base = (a0 + r * G) * N
            pltpu.sync_copy(rk_hbm.at[pl.ds(base, G * N)], rkv)
            pltpu.sync_copy(cx_hbm.at[pl.ds(base, G * N)], cxv)
            pltpu.sync_copy(cy_hbm.at[pl.ds(base, G * N)], cyv)
            pltpu.sync_copy(cz_hbm.at[pl.ds(base, G * N)], czv)

            three = jnp.full((16,), 3, jnp.int32)
            one = jnp.full((16,), 1, jnp.int32)
            nvec = jnp.full((16,), N, jnp.int32)
            n3vec = jnp.full((16,), 3 * N, jnp.int32)

            def chunk(c, carry2):
                p0 = c * 16
                pos = lane + jnp.full((16,), p0, jnp.int32)
                aoff = lax.div(pos, nvec) * n3vec
                i3 = rkv[pl.ds(p0, 16)] * three + aoff
                i3p1 = i3 + one
                i3p2 = i3p1 + one
                plsc.store_scatter(outv, [i3], cxv[pl.ds(p0, 16)])
                plsc.store_scatter(outv, [i3p1], cyv[pl.ds(p0, 16)])
                plsc.store_scatter(outv, [i3p2], czv[pl.ds(p0, 16)])
                return carry2

            lax.fori_loop(0, CH, chunk, 0)
            pltpu.sync_copy(outv, out_hbm.at[pl.ds(base * 3, G * N * 3)])
            return carry

        lax.fori_loop(0, W // G, round_body, 0)

    return k


def kernel(positions, cell, neighbors, mask, offsets, atomic_numbers):
    B, A, N = neighbors.shape
    BA = B * A
    NW = _num_workers()
    G = 64
    T = 256
    W = A // NW
    assert A % NW == 0 and W % G == 0 and (G * N) % 16 == 0 and A % T == 0
    del BA
    gather_k = _build_gather(1, A, N, NW, G)
    compute_k = _build_compute(A, A, N, T)
    scatter_k = _build_scatter(1, A, N, NW, G)
    off_p = jnp.moveaxis(offsets, 3, 0)

    outs = []
    for b in range(B):
        gx, gy, gz = gather_k(positions[b].reshape(1, A * 3),
                              neighbors[b].reshape(A * N))
        rank, cx, cy, cz = compute_k(
            gx.reshape(A, N), gy.reshape(A, N), gz.reshape(A, N),
            off_p[0, b], off_p[1, b], off_p[2, b],
            positions[b], cell[b].reshape(1, 1, 9), mask[b])
        outs.append(scatter_k(rank.reshape(-1), cx.reshape(-1),
                              cy.reshape(-1), cz.reshape(-1)))
    return jnp.stack(outs).reshape(B, A, 3 * N)


# speedup vs baseline: 35.0031x; 1.0016x over previous
import functools

import numpy as np
import jax
import jax.numpy as jnp
from jax import lax
from jax.experimental import pallas as pl
from jax.experimental.pallas import tpu as pltpu
from jax.experimental.pallas import tpu_sc as plsc

_RC = 5.0


def _num_workers():
    try:
        info = plsc.get_sparse_core_info()
        return info.num_cores * info.num_subcores
    except Exception:
        return 32


def _build_gather(B, A, N, NW, G):
    W = (B * A) // NW
    CH = (G * N) // 16
    mesh = plsc.VectorSubcoreMesh(core_axis_name="c", subcore_axis_name="s")

    @functools.partial(
        pl.kernel,
        out_type=[jax.ShapeDtypeStruct((B * A * N,), jnp.float32)] * 3,
        mesh=mesh,
        compiler_params=pltpu.CompilerParams(needs_layout_passes=False),
        scratch_types=[
            pltpu.VMEM((A * 3,), jnp.float32),
            pltpu.VMEM((G * N,), jnp.int32),
            pltpu.VMEM((G * N,), jnp.float32),
            pltpu.VMEM((G * N,), jnp.float32),
            pltpu.VMEM((G * N,), jnp.float32),
        ],
    )
    def k(pos_hbm, nb_hbm, gx_hbm, gy_hbm, gz_hbm, tab, nbv, gxs, gys, gzs):
        wid = lax.axis_index("s") * 2 + lax.axis_index("c")
        a0 = wid * W
        b = a0 // A
        pltpu.sync_copy(pos_hbm.at[b], tab)

        def round_body(r, carry):
            base = (a0 + r * G) * N
            pltpu.sync_copy(nb_hbm.at[pl.ds(base, G * N)], nbv)

            three = jnp.full((16,), 3, jnp.int32)
            one = jnp.full((16,), 1, jnp.int32)

            def chunk(c, carry2):
                p0 = c * 16
                i3 = nbv[pl.ds(p0, 16)] * three
                i3p1 = i3 + one
                i3p2 = i3p1 + one
                gxs[pl.ds(p0, 16)] = plsc.load_gather(tab, [i3])
                gys[pl.ds(p0, 16)] = plsc.load_gather(tab, [i3p1])
                gzs[pl.ds(p0, 16)] = plsc.load_gather(tab, [i3p2])
                return carry2

            lax.fori_loop(0, CH, chunk, 0)
            pltpu.sync_copy(gxs, gx_hbm.at[pl.ds(base, G * N)])
            pltpu.sync_copy(gys, gy_hbm.at[pl.ds(base, G * N)])
            pltpu.sync_copy(gzs, gz_hbm.at[pl.ds(base, G * N)])
            return carry

        lax.fori_loop(0, W // G, round_body, 0)

    return k


def _build_compute(BA, A, N, T):
    def body(gx_ref, gy_ref, gz_ref, ox_ref, oy_ref, oz_ref, ps_ref, cl_ref,
             mk_ref, rk_ref, cx_ref, cy_ref, cz_ref):
        gx, gy, gz = gx_ref[...], gy_ref[...], gz_ref[...]
        ox, oy, oz = ox_ref[...], oy_ref[...], oz_ref[...]
        px = ps_ref[:, 0:1]
        py = ps_ref[:, 1:2]
        pz = ps_ref[:, 2:3]
        cl = cl_ref[0]

        f32, bf16 = jnp.float32, jnp.bfloat16
        obx = ox.astype(bf16).astype(f32)
        oby = oy.astype(bf16).astype(f32)
        obz = oz.astype(bf16).astype(f32)
        clb = [cl[0, k].astype(bf16).astype(f32) for k in range(9)]
        ocx = obx * clb[0] + oby * clb[3] + obz * clb[6]
        ocy = obx * clb[1] + oby * clb[4] + obz * clb[7]
        ocz = obx * clb[2] + oby * clb[5] + obz * clb[8]
        dvx = (gx - px) + ocx
        dvy = (gy - py) + ocy
        dvz = (gz - pz) + ocz

        d = jnp.sqrt((dvx * dvx + dvy * dvy) + dvz * dvz)
        valid = mk_ref[...] != 0.0
        dsafe = jnp.where(valid, d, jnp.float32(1.0))
        cutf = (0.5 * (jnp.cos(dsafe * np.pi / _RC) + 1.0)
                * (dsafe < _RC).astype(jnp.float32))
        cut = jnp.where(valid, cutf / dsafe, jnp.float32(0.0))

        ii = lax.broadcasted_iota(jnp.int32, (N, N), 0)
        jj = lax.broadcasted_iota(jnp.int32, (N, N), 1)
        tri = ii < jj
        ident = (ii == jj).astype(jnp.float32)
        cutT = lax.dot_general(ident, cut, (((1,), (1,)), ((), ())),
                               preferred_element_type=jnp.float32,
                               precision=lax.Precision.HIGHEST)

        ones_row = jnp.ones((1, N), jnp.float32)
        rows = []
        for t in range(T):
            col = cutT[:, t:t + 1]
            row = cut[t:t + 1, :]
            gtm = col > row
            tiem = (col == row) & tri
            cnt = jnp.logical_or(gtm, tiem).astype(jnp.float32)
            rows.append(lax.dot_general(ones_row, cnt, (((1,), (0,)), ((), ())),
                                        preferred_element_type=jnp.float32))
        rank = jnp.concatenate(rows, axis=0)

        rk_ref[...] = rank.astype(jnp.int32)
        cx_ref[...] = cut * dvx
        cy_ref[...] = cut * dvy
        cz_ref[...] = cut * dvz

    tile = pl.BlockSpec((T, N), lambda i: (i, 0))
    return pl.pallas_call(
        body,
        grid=(BA // T,),
        in_specs=[tile, tile, tile, tile, tile, tile,
                  pl.BlockSpec((T, 3), lambda i: (i, 0)),
                  pl.BlockSpec((1, 1, 9), lambda i: ((i * T) // A, 0, 0)),
                  tile],
        out_specs=[tile, tile, tile, tile],
        out_shape=[jax.ShapeDtypeStruct((BA, N), jnp.int32)] +
                  [jax.ShapeDtypeStruct((BA, N), jnp.float32)] * 3,
    )


def _build_scatter(B, A, N, NW, G):
    W = (B * A) // NW
    CH = (G * N) // 16
    mesh = plsc.VectorSubcoreMesh(core_axis_name="c", subcore_axis_name="s")

    @functools.partial(
        pl.kernel,
        out_type=jax.ShapeDtypeStruct((B * A * N * 3,), jnp.float32),
        mesh=mesh,
        compiler_params=pltpu.CompilerParams(needs_layout_passes=False),
        scratch_types=[
            pltpu.VMEM((G * N,), jnp.int32),
            pltpu.VMEM((G * N,), jnp.float32),
            pltpu.VMEM((G * N,), jnp.float32),
            pltpu.VMEM((G * N,), jnp.float32),
            pltpu.VMEM((G * N * 3,), jnp.float32),
        ],
    )
    def k(rk_hbm, cx_hbm, cy_hbm, cz_hbm, out_hbm, rkv, cxv, cyv, czv, outv):
        wid = lax.axis_index("s") * 2 + lax.axis_index("c")
        a0 = wid * W
        lane = lax.broadcasted_iota(jnp.int32, (16,), 0)

        def round_body(r, carry):
            base = (a0 + r * G) * N
            pltpu.sync_copy(rk_hbm.at[pl.ds(base, G * N)], rkv)
            pltpu.sync_copy(cx_hbm.at[pl.ds(base, G * N)], cxv)
            pltpu.sync_copy(cy_hbm.at[pl.ds(base, G * N)], cyv)
            pltpu.sync_copy(cz_hbm.at[pl.ds(base, G * N)], czv)

            three = jnp.full((16,), 3, jnp.int32)
            one = jnp.full((16,), 1, jnp.int32)
            nvec = jnp.full((16,), N, jnp.int32)
            n3vec = jnp.full((16,), 3 * N, jnp.int32)

            def chunk(c, carry2):
                p0 = c * 16
                pos = lane + jnp.full((16,), p0, jnp.int32)
                aoff = lax.div(pos, nvec) * n3vec
                i3 = rkv[pl.ds(p0, 16)] * three + aoff
                i3p1 = i3 + one
                i3p2 = i3p1 + one
                plsc.store_scatter(outv, [i3], cxv[pl.ds(p0, 16)])
                plsc.store_scatter(outv, [i3p1], cyv[pl.ds(p0, 16)])
                plsc.store_scatter(outv, [i3p2], czv[pl.ds(p0, 16)])
                return carry2

            lax.fori_loop(0, CH, chunk, 0)
            pltpu.sync_copy(outv, out_hbm.at[pl.ds(base * 3, G * N * 3)])
            return carry

        lax.fori_loop(0, W // G, round_body, 0)

    return k


def kernel(positions, cell, neighbors, mask, offsets, atomic_numbers):
    B, A, N = neighbors.shape
    BA = B * A
    NW = _num_workers()
    GG = 128
    GS = 64
    T = 256
    W = A // NW
    assert A % NW == 0 and W % GG == 0 and W % GS == 0 and A % T == 0
    del BA
    gather_k = _build_gather(1, A, N, NW, GG)
    compute_k = _build_compute(A, A, N, T)
    scatter_k = _build_scatter(1, A, N, NW, GS)
    off_p = jnp.moveaxis(offsets, 3, 0)

    outs = []
    for b in range(B):
        gx, gy, gz = gather_k(positions[b].reshape(1, A * 3),
                              neighbors[b].reshape(A * N))
        rank, cx, cy, cz = compute_k(
            gx.reshape(A, N), gy.reshape(A, N), gz.reshape(A, N),
            off_p[0, b], off_p[1, b], off_p[2, b],
            positions[b], cell[b].reshape(1, 1, 9), mask[b])
        outs.append(scatter_k(rank.reshape(-1), cx.reshape(-1),
                              cy.reshape(-1), cz.reshape(-1)))
    return jnp.stack(outs).reshape(B, A, 3 * N)
